# baseline (speedup 1.0000x reference)
import functools

import jax
import jax.numpy as jnp
from jax import lax
from jax.experimental import pallas as pl
from jax.experimental.pallas import tpu as pltpu
from jax.experimental.pallas import tpu_sc as plsc


@functools.lru_cache(maxsize=None)
def _make_gather(B, D, rows_per_worker):
    info = plsc.get_sparse_core_info()
    num_cores, num_subcores = info.num_cores, info.num_subcores
    num_workers_used = B // rows_per_worker
    mesh = plsc.VectorSubcoreMesh(core_axis_name="c", subcore_axis_name="s")

    @functools.partial(
        pl.kernel,
        mesh=mesh,
        out_type=jax.ShapeDtypeStruct((B, D), jnp.float32),
        scratch_types=[
            pltpu.VMEM((rows_per_worker,), jnp.int32),
            pltpu.VMEM((rows_per_worker, D), jnp.float32),
            pltpu.SemaphoreType.DMA,
        ],
    )
    def gather_kernel(idx_hbm, table_hbm, out_hbm, idx_v, rows_v, sem):
        wid = lax.axis_index("s") * num_cores + lax.axis_index("c")

        @pl.when(wid < num_workers_used)
        def _():
            base = wid * rows_per_worker
            pltpu.sync_copy(idx_hbm.at[pl.ds(base, rows_per_worker)], idx_v)
            pltpu.async_copy(table_hbm.at[idx_v], rows_v, sem).wait()
            pltpu.sync_copy(rows_v, out_hbm.at[pl.ds(base, rows_per_worker)])

    return gather_kernel


def kernel(indices, table):
    D = table.shape[1]
    idx_flat = indices.reshape(-1).astype(jnp.int32)
    B = idx_flat.shape[0]
    out = _make_gather(B, D, 16)(idx_flat, table)
    return out.reshape(indices.shape + (D,))
